# baseline (speedup 1.0000x reference)
import jax
import jax.numpy as jnp
from jax import lax
from jax.experimental import pallas as pl
from jax.experimental.pallas import tpu as pltpu
from jax.experimental.pallas import tpu_sc as plsc

D_MODEL = 768
SEQ = 8192
BATCH = 4
NC, NS, L = 2, 16, 16
NW = NC * NS
JC = SEQ // NW
S = 16
NSUB = JC // S
NV = D_MODEL // L
W = S + 8
WMAX = SEQ + 1 - W

_GDN = lax.GatherDimensionNumbers(
    offset_dims=(), collapsed_slice_dims=(0,), start_index_map=(0,))


def _bcast_lane(vec, lane):
    idx = jnp.broadcast_to(lane, (L,)).astype(jnp.int32)[:, None]
    return lax.gather(vec, idx, _GDN, (1,),
                      mode=lax.GatherScatterMode.PROMISE_IN_BOUNDS)


def _sc_body(mask_hbm, table_hbm, out_hbm, mask_v, maskf_v, tbl_v, out_v,
             gsem, ssem):
    wid = lax.axis_index("s") * NC + lax.axis_index("c")
    jbase = wid * JC

    for b in range(BATCH):
        pltpu.sync_copy(mask_hbm.at[pl.ds(b * SEQ + jbase, JC)],
                        mask_v.at[pl.ds(b * JC, JC)])
    for k in range(BATCH * JC // L):
        maskf_v[pl.ds(k * L, L)] = mask_v[pl.ds(k * L, L)].astype(jnp.float32)

    def wstart(sub):
        j0 = jbase + sub * S
        return jnp.minimum(j0, (WMAX // 8) * 8)

    def gath(sub, tslot):
        return pltpu.make_async_copy(
            table_hbm.at[pl.ds(wstart(sub), W)],
            tbl_v.at[tslot, pl.ds(0, W)], gsem.at[tslot])

    def gath_start(sub, tslot):
        gath(sub, tslot).start()
        @pl.when(jbase + sub * S == SEQ - S)
        def _():
            pltpu.sync_copy(table_hbm.at[pl.ds(SEQ, 1)],
                            tbl_v.at[tslot, pl.ds(W, 1)])

    def scat_parts(b, sub):
        row0 = b * SEQ + jbase + sub * S
        return [
            pltpu.make_async_copy(out_v.at[b, pl.ds(h * (S // 2), S // 2)],
                                  out_hbm.at[pl.ds(row0 + h * (S // 2), S // 2)],
                                  ssem.at[2 * b + h])
            for h in range(2)
        ]

    def process(sub, tslot):
        for b in range(BATCH):
            @pl.when(sub > 0)
            def _():
                for p in scat_parts(b, sub - 1):
                    p.wait()

        mv = [maskf_v[pl.ds(b * JC + sub * S, L)] for b in range(BATCH)]
        roff = jbase + sub * S + 1 - wstart(sub)

        @plsc.parallel_loop(0, S, 1, unroll=2)
        def _rows(r):
            bms = [_bcast_lane(mv[b], r) for b in range(BATCH)]
            for v in range(NV):
                t = tbl_v[tslot, roff + r, pl.ds(v * L, L)]
                for b in range(BATCH):
                    out_v[b, r, pl.ds(v * L, L)] = t * bms[b]

        for b in range(BATCH):
            for p in scat_parts(b, sub):
                p.start()

    gath_start(0, 0)

    def pair_body(it, carry):
        sub0 = 2 * it
        gath(sub0, 0).wait()
        gath_start(sub0 + 1, 1)
        process(sub0, 0)
        gath(sub0 + 1, 1).wait()

        @pl.when(it + 1 < NSUB // 2)
        def _():
            gath_start(sub0 + 2, 0)
        process(sub0 + 1, 1)
        return carry

    lax.fori_loop(0, NSUB // 2, pair_body, 0)
    for b in range(BATCH):
        for p in scat_parts(b, NSUB - 1):
            p.wait()


def kernel(input_ids, mask, table):
    del input_ids
    mask_flat = mask.reshape(BATCH * SEQ).astype(jnp.int32)
    table = table.astype(jnp.float32)

    out = pl.kernel(
        _sc_body,
        out_type=jax.ShapeDtypeStruct((BATCH * SEQ, D_MODEL), jnp.float32),
        mesh=plsc.VectorSubcoreMesh(core_axis_name="c", subcore_axis_name="s"),
        scratch_types=[
            pltpu.VMEM((BATCH * JC,), jnp.int32),
            pltpu.VMEM((BATCH * JC,), jnp.float32),
            pltpu.VMEM((2, W + 1, D_MODEL), jnp.float32),
            pltpu.VMEM((BATCH, S, D_MODEL), jnp.float32),
            pltpu.SemaphoreType.DMA((2,)),
            pltpu.SemaphoreType.DMA((2 * BATCH,)),
        ],
    )(mask_flat, table)
    return out.reshape(BATCH, SEQ, D_MODEL)


# speedup vs baseline: 1.0483x; 1.0483x over previous
import jax
import jax.numpy as jnp
from jax import lax
from jax.experimental import pallas as pl
from jax.experimental.pallas import tpu as pltpu
from jax.experimental.pallas import tpu_sc as plsc

D_MODEL = 768
SEQ = 8192
BATCH = 4
NC, NS, L = 2, 16, 16
NW = NC * NS
JC = SEQ // NW
S = 16
NSUB = JC // S
NV = D_MODEL // L
W = S + 8
WMAX = SEQ + 1 - W

_GDN = lax.GatherDimensionNumbers(
    offset_dims=(), collapsed_slice_dims=(0,), start_index_map=(0,))


def _bcast_lane(vec, lane):
    idx = jnp.broadcast_to(lane, (L,)).astype(jnp.int32)[:, None]
    return lax.gather(vec, idx, _GDN, (1,),
                      mode=lax.GatherScatterMode.PROMISE_IN_BOUNDS)


def _sc_body(mask_hbm, table_hbm, out_hbm, mask_v, maskf_v, tbl_v, out_v,
             gsem, ssem):
    wid = lax.axis_index("s") * NC + lax.axis_index("c")
    jbase = wid * JC

    for b in range(BATCH):
        pltpu.sync_copy(mask_hbm.at[pl.ds(b * SEQ + jbase, JC)],
                        mask_v.at[pl.ds(b * JC, JC)])
    for k in range(BATCH * JC // L):
        maskf_v[pl.ds(k * L, L)] = mask_v[pl.ds(k * L, L)].astype(jnp.float32)

    def wstart(sub):
        j0 = jbase + sub * S
        return jnp.minimum(j0, (WMAX // 8) * 8)

    def gath(sub, tslot):
        return pltpu.make_async_copy(
            table_hbm.at[pl.ds(wstart(sub), W)],
            tbl_v.at[tslot, pl.ds(0, W)], gsem.at[tslot])

    def gath_start(sub, tslot):
        gath(sub, tslot).start()
        @pl.when(jbase + sub * S == SEQ - S)
        def _():
            pltpu.sync_copy(table_hbm.at[pl.ds(SEQ, 1)],
                            tbl_v.at[tslot, pl.ds(W, 1)])

    def scat_parts(b, sub):
        row0 = b * SEQ + jbase + sub * S
        return [
            pltpu.make_async_copy(out_v.at[b, pl.ds(h * (S // 2), S // 2)],
                                  out_hbm.at[pl.ds(row0 + h * (S // 2), S // 2)],
                                  ssem.at[2 * b + h])
            for h in range(2)
        ]

    def process(sub, tslot):
        for b in range(BATCH):
            @pl.when(sub > 0)
            def _():
                for p in scat_parts(b, sub - 1):
                    p.wait()

        mv = [maskf_v[pl.ds(b * JC + sub * S, L)] for b in range(BATCH)]
        roff = jbase + sub * S + 1 - wstart(sub)

        @plsc.parallel_loop(0, S, 1, unroll=4)
        def _rows(r):
            bms = [_bcast_lane(mv[b], r) for b in range(BATCH)]
            for v in range(NV):
                t = tbl_v[tslot, roff + r, pl.ds(v * L, L)]
                for b in range(BATCH):
                    out_v[b, r, pl.ds(v * L, L)] = t * bms[b]

        for b in range(BATCH):
            for p in scat_parts(b, sub):
                p.start()

    gath_start(0, 0)

    def pair_body(it, carry):
        sub0 = 2 * it
        gath(sub0, 0).wait()
        gath_start(sub0 + 1, 1)
        process(sub0, 0)
        gath(sub0 + 1, 1).wait()

        @pl.when(it + 1 < NSUB // 2)
        def _():
            gath_start(sub0 + 2, 0)
        process(sub0 + 1, 1)
        return carry

    lax.fori_loop(0, NSUB // 2, pair_body, 0)
    for b in range(BATCH):
        for p in scat_parts(b, NSUB - 1):
            p.wait()


def kernel(input_ids, mask, table):
    del input_ids
    mask_flat = mask.reshape(BATCH * SEQ).astype(jnp.int32)
    table = table.astype(jnp.float32)

    out = pl.kernel(
        _sc_body,
        out_type=jax.ShapeDtypeStruct((BATCH * SEQ, D_MODEL), jnp.float32),
        mesh=plsc.VectorSubcoreMesh(core_axis_name="c", subcore_axis_name="s"),
        scratch_types=[
            pltpu.VMEM((BATCH * JC,), jnp.int32),
            pltpu.VMEM((BATCH * JC,), jnp.float32),
            pltpu.VMEM((2, W + 1, D_MODEL), jnp.float32),
            pltpu.VMEM((BATCH, S, D_MODEL), jnp.float32),
            pltpu.SemaphoreType.DMA((2,)),
            pltpu.SemaphoreType.DMA((2 * BATCH,)),
        ],
    )(mask_flat, table)
    return out.reshape(BATCH, SEQ, D_MODEL)


# speedup vs baseline: 1.0848x; 1.0348x over previous
import jax
import jax.numpy as jnp
from jax import lax
from jax.experimental import pallas as pl
from jax.experimental.pallas import tpu as pltpu
from jax.experimental.pallas import tpu_sc as plsc

D_MODEL = 768
SEQ = 8192
BATCH = 4
NC, NS, L = 2, 16, 16
NW = NC * NS
JC = SEQ // NW
S = 32
NSUB = JC // S
NV = D_MODEL // L
W = S + 8
WMAX = SEQ + 1 - W

_GDN = lax.GatherDimensionNumbers(
    offset_dims=(), collapsed_slice_dims=(0,), start_index_map=(0,))


def _bcast_lane(vec, lane):
    idx = jnp.broadcast_to(lane, (L,)).astype(jnp.int32)[:, None]
    return lax.gather(vec, idx, _GDN, (1,),
                      mode=lax.GatherScatterMode.PROMISE_IN_BOUNDS)


def _sc_body(mask_hbm, table_hbm, out_hbm, mask_v, maskf_v, tbl_v, out_v,
             gsem, ssem):
    wid = lax.axis_index("s") * NC + lax.axis_index("c")
    jbase = wid * JC

    for b in range(BATCH):
        pltpu.sync_copy(mask_hbm.at[pl.ds(b * SEQ + jbase, JC)],
                        mask_v.at[pl.ds(b * JC, JC)])
    for k in range(BATCH * JC // L):
        maskf_v[pl.ds(k * L, L)] = mask_v[pl.ds(k * L, L)].astype(jnp.float32)

    def wstart(sub):
        j0 = jbase + sub * S
        return jnp.minimum(j0, (WMAX // 8) * 8)

    def gath(sub, tslot):
        return pltpu.make_async_copy(
            table_hbm.at[pl.ds(wstart(sub), W)],
            tbl_v.at[tslot, pl.ds(0, W)], gsem.at[tslot])

    def gath_start(sub, tslot):
        gath(sub, tslot).start()
        @pl.when(jbase + sub * S == SEQ - S)
        def _():
            pltpu.sync_copy(table_hbm.at[pl.ds(SEQ, 1)],
                            tbl_v.at[tslot, pl.ds(W, 1)])

    def scat_parts(b, sub):
        row0 = b * SEQ + jbase + sub * S
        return [
            pltpu.make_async_copy(out_v.at[b % 2, pl.ds(h * (S // 2), S // 2)],
                                  out_hbm.at[pl.ds(row0 + h * (S // 2), S // 2)],
                                  ssem.at[2 * b + h])
            for h in range(2)
        ]

    def process(sub, tslot):
        mv = [maskf_v[pl.ds(b * JC + sub * S, L)] for b in range(BATCH)]
        mv2 = [maskf_v[pl.ds(b * JC + sub * S + L, L)] for b in range(BATCH)]
        roff = jbase + sub * S + 1 - wstart(sub)

        for b in range(BATCH):
            if b >= 2:
                for p in scat_parts(b - 2, sub):
                    p.wait()
            else:
                @pl.when(sub > 0)
                def _():
                    for p in scat_parts(b + 2, sub - 1):
                        p.wait()

            @plsc.parallel_loop(0, S, 1, unroll=2)
            def _rows(r):
                lane = r & (L - 1)
                mvec = jnp.where(r < L, mv[b], mv2[b])
                bm = _bcast_lane(mvec, lane)
                for v in range(NV):
                    t = tbl_v[tslot, roff + r, pl.ds(v * L, L)]
                    out_v[b % 2, r, pl.ds(v * L, L)] = t * bm

            for p in scat_parts(b, sub):
                p.start()

    gath_start(0, 0)

    def pair_body(it, carry):
        sub0 = 2 * it
        gath(sub0, 0).wait()
        gath_start(sub0 + 1, 1)
        process(sub0, 0)
        gath(sub0 + 1, 1).wait()

        @pl.when(it + 1 < NSUB // 2)
        def _():
            gath_start(sub0 + 2, 0)
        process(sub0 + 1, 1)
        return carry

    lax.fori_loop(0, NSUB // 2, pair_body, 0)
    for b in range(2, BATCH):
        for p in scat_parts(b, NSUB - 1):
            p.wait()


def kernel(input_ids, mask, table):
    del input_ids
    mask_flat = mask.reshape(BATCH * SEQ).astype(jnp.int32)
    table = table.astype(jnp.float32)

    out = pl.kernel(
        _sc_body,
        out_type=jax.ShapeDtypeStruct((BATCH * SEQ, D_MODEL), jnp.float32),
        mesh=plsc.VectorSubcoreMesh(core_axis_name="c", subcore_axis_name="s"),
        scratch_types=[
            pltpu.VMEM((BATCH * JC,), jnp.int32),
            pltpu.VMEM((BATCH * JC,), jnp.float32),
            pltpu.VMEM((2, W + 1, D_MODEL), jnp.float32),
            pltpu.VMEM((2, S, D_MODEL), jnp.float32),
            pltpu.SemaphoreType.DMA((2,)),
            pltpu.SemaphoreType.DMA((2 * BATCH,)),
        ],
    )(mask_flat, table)
    return out.reshape(BATCH, SEQ, D_MODEL)
